# baseline (speedup 1.0000x reference)
import functools

import jax
import jax.numpy as jnp
from jax import lax
from jax.experimental import pallas as pl
from jax.experimental.pallas import tpu as pltpu
from jax.experimental.pallas import tpu_sc as plsc

BATCH = 4096
D = 64
W = 2 * D
L = 16
NC = 2
NS = 16
NW = NC * NS
BPW = BATCH // NW
MARGIN = 1.0

_MESH = plsc.VectorSubcoreMesh(core_axis_name="c", subcore_axis_name="s")


@functools.partial(
    pl.kernel,
    out_type=jax.ShapeDtypeStruct((NW * L,), jnp.float32),
    mesh=_MESH,
    compiler_params=pltpu.CompilerParams(needs_layout_passes=False),
    scratch_types=[
        pltpu.VMEM((BPW,), jnp.int32),
        pltpu.VMEM((BPW,), jnp.int32),
        pltpu.VMEM((BPW,), jnp.int32),
        pltpu.VMEM((BPW,), jnp.int32),
        pltpu.VMEM((BPW,), jnp.int32),
        pltpu.VMEM((BPW,), jnp.int32),
        pltpu.VMEM((BPW,), jnp.int32),
        pltpu.VMEM((BPW,), jnp.int32),
        pltpu.VMEM((BPW,), jnp.int32),
        pltpu.VMEM((BPW,), jnp.int32),
        pltpu.VMEM((BPW,), jnp.int32),
        pltpu.VMEM((BPW,), jnp.int32),
        pltpu.VMEM((BPW, W), jnp.float32),
        pltpu.VMEM((BPW, W), jnp.float32),
        pltpu.VMEM((BPW, W), jnp.float32),
        pltpu.VMEM((BPW, W), jnp.float32),
        pltpu.VMEM((BPW, W), jnp.float32),
        pltpu.VMEM((BPW, W), jnp.float32),
        pltpu.VMEM((L,), jnp.float32),
        pltpu.SemaphoreType.DMA,
    ],
)
def _transe_sc(ph_h, pt_h, pr_h, nh_h, nt_h, nr_h, ent_h, rel_h, out_h,
               ph_i, pt_i, pr_i, nh_i, nt_i, nr_i,
               ph_j, pt_j, pr_j, nh_j, nt_j, nr_j,
               ph_r, pt_r, pr_r, nh_r, nt_r, nr_r,
               res_v, sem):
    wid = lax.axis_index("s") * NC + lax.axis_index("c")
    base = wid * BPW

    idx_refs = (ph_i, pt_i, pr_i, nh_i, nt_i, nr_i)
    half_refs = (ph_j, pt_j, pr_j, nh_j, nt_j, nr_j)
    for src, dst in zip((ph_h, pt_h, pr_h, nh_h, nt_h, nr_h), idx_refs):
        pltpu.sync_copy(src.at[pl.ds(base, BPW)], dst)
    for iref, jref in zip(idx_refs, half_refs):
        for k in range(BPW // L):
            sl = pl.ds(k * L, L)
            jref[sl] = lax.shift_right_logical(iref[sl], 1)

    copies = [
        pltpu.async_copy(ent_h.at[ph_j], ph_r, sem),
        pltpu.async_copy(ent_h.at[pt_j], pt_r, sem),
        pltpu.async_copy(rel_h.at[pr_j], pr_r, sem),
        pltpu.async_copy(ent_h.at[nh_j], nh_r, sem),
        pltpu.async_copy(ent_h.at[nt_j], nt_r, sem),
        pltpu.async_copy(rel_h.at[nr_j], nr_r, sem),
    ]
    for c in copies:
        c.wait()

    lanes = lax.iota(jnp.int32, 16)

    def group_body(g, tot):
        sl = pl.ds(g * L, L)
        offv = [(iref[sl] & 1) * D for iref in idx_refs]
        for j in range(L):
            i = g * L + j
            o = [ov[j] for ov in offv]
            accp = jnp.zeros((L,), jnp.float32)
            accn = jnp.zeros((L,), jnp.float32)
            for c in range(D // L):
                cb = c * L
                accp = accp + jnp.abs(ph_r[i, pl.ds(o[0] + cb, L)]
                                      - pt_r[i, pl.ds(o[1] + cb, L)]
                                      + pr_r[i, pl.ds(o[2] + cb, L)])
                accn = accn + jnp.abs(nh_r[i, pl.ds(o[3] + cb, L)]
                                      - nt_r[i, pl.ds(o[4] + cb, L)]
                                      + nr_r[i, pl.ds(o[5] + cb, L)])
            p = jnp.sum(accp)
            n = jnp.sum(accn)
            tot = tot + jnp.maximum(p - n + MARGIN, 0.0)
        return tot

    tot = lax.fori_loop(0, BPW // L, group_body, jnp.float32(0.0))
    res_v[...] = jnp.where(lanes == 0, tot, 0.0)
    pltpu.sync_copy(res_v, out_h.at[pl.ds(wid * L, L)])


def kernel(pos_h, pos_t, pos_r, neg_h, neg_t, neg_r, ent_embeddings, rel_embeddings):
    idx = [x.reshape(-1).astype(jnp.int32)
           for x in (pos_h, pos_t, pos_r, neg_h, neg_t, neg_r)]
    ent2 = ent_embeddings.reshape(-1, W)
    rel2 = rel_embeddings.reshape(-1, W)
    partials = _transe_sc(*idx, ent2, rel2)
    return jnp.sum(partials)
